# baseline (speedup 1.0000x reference)
import jax
import jax.numpy as jnp
from jax import lax
from jax.experimental import pallas as pl
from jax.experimental.pallas import tpu as pltpu
from jax.experimental.pallas import tpu_sc as plsc

B, S, D = 4, 4096, 1024
H = S // 2
CC = 128
NBUF = 4
NCHC = H // CC
NCH = 2 * NCHC
NG = NCH // NBUF


def _shuffle_body(mem_in, out, buf0, buf1, buf2, buf3,
                  sin0, sin1, sin2, sin3, sout0, sout1, sout2, sout3):
    mem_hbm = mem_in.reshape(B, 2, H, D)
    cid = lax.axis_index("c")

    bufs = (buf0, buf1, buf2, buf3)
    sin = (sin0, sin1, sin2, sin3)
    sout = (sout0, sout1, sout2, sout3)

    def bk(c):
        return 2 * cid + c // NCHC, (c % NCHC) * CC

    def start_in(j, c):
        bb, k = bk(c)
        pltpu.async_copy(mem_hbm.at[bb, 0, pl.ds(k, CC), :],
                         bufs[j].at[:, 0, :], sin[j])
        pltpu.async_copy(mem_hbm.at[bb, 1, pl.ds(k, CC), :],
                         bufs[j].at[:, 1, :], sin[j])

    def wait_in(j):
        pltpu.make_async_copy(mem_hbm.at[0, 0, pl.ds(0, CC), :],
                              bufs[j].at[:, 0, :], sin[j]).wait()
        pltpu.make_async_copy(mem_hbm.at[0, 1, pl.ds(0, CC), :],
                              bufs[j].at[:, 1, :], sin[j]).wait()

    def start_out(j, c):
        bb, k = bk(c)
        pltpu.async_copy(bufs[j].reshape(2 * CC, D),
                         out.at[bb, pl.ds(2 * k, 2 * CC), :], sout[j])

    def wait_out(j):
        pltpu.make_async_copy(bufs[j].reshape(2 * CC, D),
                              out.at[0, pl.ds(0, 2 * CC), :], sout[j]).wait()

    for j in range(NBUF - 1):
        start_in(j, j)

    def ring_round(i, carry):
        for jj in range(NBUF):
            c = i * NBUF + jj
            jw = (jj + NBUF - 1) % NBUF

            @pl.when(c >= 1)
            def _():
                wait_out(jw)

            @pl.when(c + NBUF - 1 < NCH)
            def _():
                start_in(jw, c + NBUF - 1)

            wait_in(jj)
            start_out(jj, c)
        return carry

    lax.fori_loop(0, NG, ring_round, 0)
    wait_out((NCH - 1) % NBUF)


def kernel(mem):
    return pl.kernel(
        _shuffle_body,
        out_type=jax.ShapeDtypeStruct((B, S, D), jnp.float32),
        mesh=plsc.ScalarSubcoreMesh(axis_name="c", num_cores=2),
        scratch_types=(
            [pltpu.VMEM_SHARED((CC, 2, D), jnp.float32)] * NBUF
            + [pltpu.SemaphoreType.DMA] * (2 * NBUF)
        ),
    )(mem)


# speedup vs baseline: 1.1577x; 1.1577x over previous
import jax
import jax.numpy as jnp
from jax import lax
from jax.experimental import pallas as pl
from jax.experimental.pallas import tpu as pltpu
from jax.experimental.pallas import tpu_sc as plsc
from jax._src.pallas import mpmd
from jax._src.pallas import core as pallas_core
from jax._src.pallas.mosaic import core as tpu_core

B, S, D = 4, 4096, 1024
H = S // 2

T_TEC = 2048
NSUB = 32
RCHUNKS = NSUB // B
RPS = T_TEC // 2 // RCHUNKS
C = 8
NBUF = 4
NCH = RPS // C
NG = NCH // NBUF

HS0 = T_TEC // 2
CC = 256
SNBUF = 2
NCHC = (H - HS0) // CC
SNCH = 2 * NCHC
SNG = SNCH // SNBUF


def _tec_body(mem_in, out, buf0, buf1, buf2, buf3,
              sin0, sin1, sin2, sin3, sout0, sout1, sout2, sout3,
              sbuf0, sbuf1, ssin0, ssin1, ssout0, ssout1):
    mem_hbm = mem_in.reshape(B, 2, H, D)
    nc = 2
    wid = lax.axis_index("s") * nc + lax.axis_index("c")
    b = wid % B
    r0 = (wid // B) * RPS

    bufs = (buf0, buf1, buf2, buf3)
    sin = (sin0, sin1, sin2, sin3)
    sout = (sout0, sout1, sout2, sout3)

    def start_in(j, c):
        k = r0 + c * C
        pltpu.async_copy(mem_hbm.at[b, 0, pl.ds(k, C), :],
                         bufs[j].at[:, 0, :], sin[j])
        pltpu.async_copy(mem_hbm.at[b, 1, pl.ds(k, C), :],
                         bufs[j].at[:, 1, :], sin[j])

    def wait_in(j):
        pltpu.make_async_copy(mem_hbm.at[b, 0, pl.ds(r0, C), :],
                              bufs[j].at[:, 0, :], sin[j]).wait()
        pltpu.make_async_copy(mem_hbm.at[b, 1, pl.ds(r0, C), :],
                              bufs[j].at[:, 1, :], sin[j]).wait()

    def start_out(j, c):
        k = r0 + c * C
        pltpu.async_copy(bufs[j].reshape(2 * C, D),
                         out.at[b, pl.ds(2 * k, 2 * C), :], sout[j])

    def wait_out(j):
        pltpu.make_async_copy(bufs[j].reshape(2 * C, D),
                              out.at[b, pl.ds(2 * r0, 2 * C), :],
                              sout[j]).wait()

    for j in range(NBUF - 1):
        start_in(j, j)

    def ring_round(i, carry):
        for jj in range(NBUF):
            c = i * NBUF + jj
            jw = (jj + NBUF - 1) % NBUF

            @pl.when(c >= 1)
            def _():
                wait_out(jw)

            @pl.when(c + NBUF - 1 < NCH)
            def _():
                start_in(jw, c + NBUF - 1)

            wait_in(jj)
            start_out(jj, c)
        return carry

    lax.fori_loop(0, NG, ring_round, 0)
    wait_out((NCH - 1) % NBUF)


def _scs_body(mem_in, out, buf0, buf1, buf2, buf3,
              sin0, sin1, sin2, sin3, sout0, sout1, sout2, sout3,
              sbuf0, sbuf1, ssin0, ssin1, ssout0, ssout1):
    mem_hbm = mem_in.reshape(B, 2, H, D)
    cid = lax.axis_index("c")

    bufs = (sbuf0, sbuf1)
    sin = (ssin0, ssin1)
    sout = (ssout0, ssout1)

    def bk(c):
        return 2 * cid + c // NCHC, HS0 + (c % NCHC) * CC

    def start_in(j, c):
        bb, k = bk(c)
        pltpu.async_copy(mem_hbm.at[bb, 0, pl.ds(k, CC), :],
                         bufs[j].at[:, 0, :], sin[j])
        pltpu.async_copy(mem_hbm.at[bb, 1, pl.ds(k, CC), :],
                         bufs[j].at[:, 1, :], sin[j])

    def wait_in(j):
        pltpu.make_async_copy(mem_hbm.at[0, 0, pl.ds(0, CC), :],
                              bufs[j].at[:, 0, :], sin[j]).wait()
        pltpu.make_async_copy(mem_hbm.at[0, 1, pl.ds(0, CC), :],
                              bufs[j].at[:, 1, :], sin[j]).wait()

    def start_out(j, c):
        bb, k = bk(c)
        pltpu.async_copy(bufs[j].reshape(2 * CC, D),
                         out.at[bb, pl.ds(2 * k, 2 * CC), :], sout[j])

    def wait_out(j):
        pltpu.make_async_copy(bufs[j].reshape(2 * CC, D),
                              out.at[0, pl.ds(0, 2 * CC), :], sout[j]).wait()

    for j in range(SNBUF - 1):
        start_in(j, j)

    def ring_round(i, carry):
        for jj in range(SNBUF):
            c = i * SNBUF + jj
            jw = (jj + SNBUF - 1) % SNBUF

            @pl.when(c >= 1)
            def _():
                wait_out(jw)

            @pl.when(c + SNBUF - 1 < SNCH)
            def _():
                start_in(jw, c + SNBUF - 1)

            wait_in(jj)
            start_out(jj, c)
        return carry

    lax.fori_loop(0, SNG, ring_round, 0)
    wait_out((SNCH - 1) % SNBUF)


def kernel(mem):
    vec_mesh = plsc.VectorSubcoreMesh(core_axis_name="c", subcore_axis_name="s")
    scs_mesh = plsc.ScalarSubcoreMesh(axis_name="c", num_cores=2)
    tec_vmem = pallas_core.CoreMemorySpace(tpu_core.MemorySpace.VMEM, vec_mesh)
    return mpmd.mpmd_map(
        [(scs_mesh, _scs_body), (vec_mesh, _tec_body)],
        out_types=jax.ShapeDtypeStruct((B, S, D), jnp.float32),
        scratch_types=(
            [tec_vmem((C, 2, D), jnp.float32)] * NBUF
            + [pltpu.SemaphoreType.DMA @ vec_mesh] * (2 * NBUF)
            + [pltpu.VMEM_SHARED((CC, 2, D), jnp.float32)] * SNBUF
            + [pltpu.SemaphoreType.DMA @ scs_mesh] * (2 * SNBUF)
        ),
    )(mem)
